# Initial kernel scaffold; baseline (speedup 1.0000x reference)
import jax
import jax.numpy as jnp
from jax.experimental import pallas as pl


def kernel(x, edge_index, norm, W1, b1, W2, b2, Wp, bp):
    raise NotImplementedError("write your pallas kernel here")


# speedup vs baseline: 10.2220x; 10.2220x over previous
import functools

import jax
import jax.numpy as jnp
from jax import lax
from jax.experimental import pallas as pl
from jax.experimental.pallas import tpu as pltpu
from jax.experimental.pallas import tpu_sc as plsc

L = 16
CHUNK = 128
G = 8

_GD = jax.lax.GatherDimensionNumbers(
    offset_dims=(), collapsed_slice_dims=(0,), start_index_map=(0,))


def _bcast16(vec, e):
  idx = jnp.full((16, 1), e, jnp.int32)
  return jax.lax.gather(
      vec, idx, _GD, (1,),
      mode=jax.lax.GatherScatterMode.PROMISE_IN_BOUNDS)
NCORES = 2
NSUB = 16
NW = NCORES * NSUB
KSTEPS = 10


@functools.partial(jax.jit, static_argnames=("n_pad", "cpt"))
def _sc_round(cur, srcr, dstr, nrmr, *, n_pad, cpt):
  mesh = plsc.VectorSubcoreMesh(core_axis_name="c", subcore_axis_name="s")
  hr = n_pad // 2
  stripe = hr // NSUB
  zrows = stripe // 8
  nblk = cpt // G

  @functools.partial(
      pl.kernel,
      out_type=jax.ShapeDtypeStruct((NCORES, hr, L), jnp.float32),
      mesh=mesh,
      compiler_params=pltpu.CompilerParams(use_tc_tiling_on_sc=False),
      scratch_types=[
          pltpu.VMEM((2, G, CHUNK), jnp.int32),
          pltpu.VMEM((2, G, CHUNK), jnp.int32),
          pltpu.VMEM((2, G, CHUNK), jnp.float32),
          pltpu.VMEM((2, G, CHUNK, L), jnp.float32),
          pltpu.VMEM((zrows, L), jnp.float32),
          pltpu.VMEM_SHARED((hr, L), jnp.float32),
          pltpu.SemaphoreType.DMA,
          pltpu.SemaphoreType.DMA,
          pltpu.SemaphoreType.DMA,
          pltpu.SemaphoreType.DMA,
          pltpu.SemaphoreType.DMA,
          pltpu.SemaphoreType.DMA,
      ],
  )
  def round_kernel(cur_h, src_h, dst_h, nrm_h, out_h,
                   src_v, dst_v, nrm_s, rows_v, zbuf, acc,
                   isem0, isem1, gsem0, gsem1, ssem0, ssem1):
    cid = lax.axis_index("c")
    sid = lax.axis_index("s")
    isem = (isem0, isem1)
    gsem = (gsem0, gsem1)
    ssem = (ssem0, ssem1)
    tile_base = sid * cpt

    @pl.loop(0, zrows)
    def _(i):
      zbuf[i, :] = jnp.zeros((L,), jnp.float32)

    for j in range(8):
      pltpu.sync_copy(zbuf, acc.at[pl.ds(sid * stripe + j * zrows, zrows)])
    plsc.subcore_barrier()

    def issue_idx(buf, blk):
      base = tile_base + blk * G
      pltpu.async_copy(src_h.at[pl.ds(base, G)], src_v.at[buf], isem[buf])
      pltpu.async_copy(dst_h.at[pl.ds(base, G)], dst_v.at[buf], isem[buf])
      pltpu.async_copy(nrm_h.at[pl.ds(base, G)], nrm_s.at[buf], isem[buf])

    def wait_idx(buf, blk):
      base = tile_base + blk * G
      pltpu.make_async_copy(src_h.at[pl.ds(base, G)], src_v.at[buf],
                            isem[buf]).wait()
      pltpu.make_async_copy(dst_h.at[pl.ds(base, G)], dst_v.at[buf],
                            isem[buf]).wait()
      pltpu.make_async_copy(nrm_h.at[pl.ds(base, G)], nrm_s.at[buf],
                            isem[buf]).wait()

    def issue_gathers(buf):
      for g in range(G):
        pltpu.async_copy(cur_h.at[src_v.at[buf, g]], rows_v.at[buf, g],
                         gsem[buf])

    def wait_gather(buf, g):
      pltpu.make_async_copy(cur_h.at[src_v.at[buf, g]], rows_v.at[buf, g],
                            gsem[buf]).wait()

    issue_idx(0, 0)
    wait_idx(0, 0)
    issue_gathers(0)
    issue_idx(1, 1)

    @pl.loop(0, nblk, step=2)
    def _(blk0):
      for half in range(2):
        blk = blk0 + half
        buf = half
        nbuf = 1 - half

        @pl.when(blk + 1 < nblk)
        def _():
          wait_idx(nbuf, blk + 1)
          issue_gathers(nbuf)

        handles = []
        for g in range(G):
          wait_gather(buf, g)

          @pl.loop(0, CHUNK, step=16)
          def _(e0, _g=g, _buf=buf):
            d16 = dst_v[_buf, _g, pl.ds(e0, 16)]
            n16 = nrm_s[_buf, _g, pl.ds(e0, 16)]
            dl = d16 - cid * hr
            ok = (dl >= 0) & (dl < hr)
            dst_v[_buf, _g, pl.ds(e0, 16)] = jnp.clip(dl, 0, hr - 1)
            f16 = jnp.where(ok, n16, 0.0)
            for e in range(16):
              ns = _bcast16(f16, e)
              rows_v[_buf, _g, e0 + e, :] = rows_v[_buf, _g, e0 + e, :] * ns

          handles.append(
              pltpu.async_copy(rows_v.at[buf, g], acc.at[dst_v.at[buf, g]],
                               ssem[buf], add=True))
        for h in handles:
          h.wait()

        @pl.when(blk + 2 < nblk)
        def _():
          issue_idx(buf, blk + 2)

    plsc.subcore_barrier()
    for j in range(8):
      off = sid * stripe + j * zrows
      pltpu.sync_copy(acc.at[pl.ds(off, zrows)], zbuf)
      pltpu.sync_copy(zbuf, out_h.at[cid].at[pl.ds(off, zrows)])


  return round_kernel(cur, srcr, dstr, nrmr)


def _tc_mlp(x_p, W1, b1r, W2p, b2r, params, n_pad):
  bn = n_pad // 16
  f_in = x_p.shape[1]
  hid = W1.shape[1]

  def body(x_ref, w1_ref, b1_ref, w2_ref, b2_ref, p_ref, h_ref, acc_ref):
    xb = x_ref[...]
    z = jnp.dot(xb, w1_ref[...], preferred_element_type=jnp.float32)
    z = jnp.maximum(z + b1_ref[...], 0.0)
    h = jnp.dot(z, w2_ref[...], preferred_element_type=jnp.float32)
    h = h + b2_ref[...]
    wp = p_ref[0:1, :]
    bp = p_ref[1:2, 0:1]
    s = jnp.sum(h * wp, axis=1, keepdims=True) + bp
    r = jax.nn.sigmoid(s)
    h_ref[...] = h
    acc_ref[...] = r * h

  return pl.pallas_call(
      body,
      grid=(n_pad // bn,),
      in_specs=[
          pl.BlockSpec((bn, f_in), lambda i: (i, 0)),
          pl.BlockSpec((f_in, hid), lambda i: (0, 0)),
          pl.BlockSpec((1, hid), lambda i: (0, 0)),
          pl.BlockSpec((hid, L), lambda i: (0, 0)),
          pl.BlockSpec((1, L), lambda i: (0, 0)),
          pl.BlockSpec((2, L), lambda i: (0, 0)),
      ],
      out_specs=[
          pl.BlockSpec((bn, L), lambda i: (i, 0)),
          pl.BlockSpec((bn, L), lambda i: (i, 0)),
      ],
      out_shape=[
          jax.ShapeDtypeStruct((n_pad, L), jnp.float32),
          jax.ShapeDtypeStruct((n_pad, L), jnp.float32),
      ],
  )(x_p, W1, b1r, W2p, b2r, params)


def _tc_combine(parts, acc_in, params, n_pad, c_real, last):
  bn = n_pad // 16
  nxt = parts.reshape(n_pad, L)

  def body(p_ref, a_ref, prm_ref, aout_ref):
    h = p_ref[...]
    wp = prm_ref[0:1, :]
    bp = prm_ref[1:2, 0:1]
    s = jnp.sum(h * wp, axis=1, keepdims=True) + bp
    r = jax.nn.sigmoid(s)
    a = a_ref[...] + r * h
    if last:
      ch = lax.broadcasted_iota(jnp.int32, (bn, L), 1)
      mask = ch < c_real
      am = jnp.where(mask, a, -1e30)
      m = jnp.max(am, axis=1, keepdims=True)
      ex = jnp.where(mask, jnp.exp(a - m), 0.0)
      lse = jnp.log(jnp.sum(ex, axis=1, keepdims=True))
      aout_ref[...] = a - m - lse
    else:
      aout_ref[...] = a

  return pl.pallas_call(
      body,
      grid=(n_pad // bn,),
      in_specs=[
          pl.BlockSpec((bn, L), lambda i: (i, 0)),
          pl.BlockSpec((bn, L), lambda i: (i, 0)),
          pl.BlockSpec((2, L), lambda i: (0, 0)),
      ],
      out_specs=[
          pl.BlockSpec((bn, L), lambda i: (i, 0)),
      ],
      out_shape=[
          jax.ShapeDtypeStruct((n_pad, L), jnp.float32),
      ],
  )(nxt, acc_in, params)[0]


def kernel(x, edge_index, norm, W1, b1, W2, b2, Wp, bp):
  n, f_in = x.shape
  e = edge_index.shape[1]
  hid = W1.shape[1]
  c = W2.shape[1]

  n_pad = -(-(n + 64) // (NSUB * 8 * 8)) * (NSUB * 8 * 8)
  cpt = -(-e // (NSUB * CHUNK * 2 * G)) * (2 * G)
  e_pad = NSUB * cpt * CHUNK
  pad = e_pad - e

  src = jnp.concatenate([edge_index[0], jnp.zeros((pad,), jnp.int32)])
  dst = jnp.concatenate(
      [edge_index[1], n + (jnp.arange(pad, dtype=jnp.int32) % 64)])
  nrm = jnp.concatenate([norm, jnp.zeros((pad,), jnp.float32)])
  srcr = src.reshape(NSUB * cpt, CHUNK)
  dstr = dst.reshape(NSUB * cpt, CHUNK)
  nrmr = nrm.reshape(NSUB * cpt, CHUNK)

  x_p = jnp.concatenate([x, jnp.zeros((n_pad - n, f_in), jnp.float32)])
  W2p = jnp.concatenate([W2, jnp.zeros((hid, L - c), jnp.float32)], axis=1)
  b2r = jnp.concatenate([b2, jnp.zeros((L - c,), jnp.float32)]).reshape(1, L)
  b1r = b1.reshape(1, hid)
  wp_pad = jnp.concatenate([Wp[:, 0], jnp.zeros((L - c,), jnp.float32)])
  bp_row = jnp.concatenate([bp, jnp.zeros((L - 1,), jnp.float32)])
  params = jnp.stack([wp_pad, bp_row])

  cur, acc = _tc_mlp(x_p, W1, b1r, W2p, b2r, params, n_pad)
  for r in range(KSTEPS):
    parts = _sc_round(cur, srcr, dstr, nrmr, n_pad=n_pad, cpt=cpt)
    cur = parts.reshape(n_pad, L)
    acc = _tc_combine(parts, acc, params, n_pad, c,
                      last=(r == KSTEPS - 1))
  return acc[:n, :c]


# speedup vs baseline: 35.8962x; 3.5117x over previous
import functools

import jax
import jax.numpy as jnp
from jax import lax
from jax.experimental import pallas as pl
from jax.experimental.pallas import tpu as pltpu
from jax.experimental.pallas import tpu_sc as plsc

L = 16
CHUNK = 128
G = 8

_GD = jax.lax.GatherDimensionNumbers(
    offset_dims=(), collapsed_slice_dims=(0,), start_index_map=(0,))


def _bcast16(vec, e):
  idx = jnp.full((16, 1), e, jnp.int32)
  return jax.lax.gather(
      vec, idx, _GD, (1,),
      mode=jax.lax.GatherScatterMode.PROMISE_IN_BOUNDS)
NCORES = 2
NSUB = 16
NW = NCORES * NSUB
KSTEPS = 10


@functools.partial(jax.jit, static_argnames=("n_pad", "cpt"))
def _sc_round(cur, srcr, dstr, nrmr, *, n_pad, cpt):
  mesh = plsc.VectorSubcoreMesh(core_axis_name="c", subcore_axis_name="s")
  hr = n_pad // 2
  stripe = hr // NSUB
  zrows = stripe // 8
  nblk = cpt // G

  @functools.partial(
      pl.kernel,
      out_type=jax.ShapeDtypeStruct((NCORES, hr, L), jnp.float32),
      mesh=mesh,
      compiler_params=pltpu.CompilerParams(use_tc_tiling_on_sc=False),
      scratch_types=[
          pltpu.VMEM((2, G, CHUNK), jnp.int32),
          pltpu.VMEM((2, G, CHUNK), jnp.int32),
          pltpu.VMEM((2, G, CHUNK), jnp.float32),
          pltpu.VMEM((2, G, CHUNK, L), jnp.float32),
          pltpu.VMEM((zrows, L), jnp.float32),
          pltpu.VMEM_SHARED((hr, L), jnp.float32),
          pltpu.SemaphoreType.DMA,
          pltpu.SemaphoreType.DMA,
          pltpu.SemaphoreType.DMA,
          pltpu.SemaphoreType.DMA,
          pltpu.SemaphoreType.DMA,
          pltpu.SemaphoreType.DMA,
      ],
  )
  def round_kernel(cur_h, src_h, dst_h, nrm_h, out_h,
                   src_v, dst_v, nrm_s, rows_v, zbuf, acc,
                   isem0, isem1, gsem0, gsem1, ssem0, ssem1):
    cid = lax.axis_index("c")
    sid = lax.axis_index("s")
    isem = (isem0, isem1)
    gsem = (gsem0, gsem1)
    ssem = (ssem0, ssem1)
    tile_base = sid * cpt

    @pl.loop(0, zrows)
    def _(i):
      zbuf[i, :] = jnp.zeros((L,), jnp.float32)

    for j in range(8):
      pltpu.sync_copy(zbuf, acc.at[pl.ds(sid * stripe + j * zrows, zrows)])
    plsc.subcore_barrier()

    def issue_idx(buf, blk):
      base = tile_base + blk * G
      pltpu.async_copy(src_h.at[pl.ds(base, G)], src_v.at[buf], isem[buf])
      pltpu.async_copy(dst_h.at[pl.ds(base, G)], dst_v.at[buf], isem[buf])
      pltpu.async_copy(nrm_h.at[pl.ds(base, G)], nrm_s.at[buf], isem[buf])

    def wait_idx(buf, blk):
      base = tile_base + blk * G
      pltpu.make_async_copy(src_h.at[pl.ds(base, G)], src_v.at[buf],
                            isem[buf]).wait()
      pltpu.make_async_copy(dst_h.at[pl.ds(base, G)], dst_v.at[buf],
                            isem[buf]).wait()
      pltpu.make_async_copy(nrm_h.at[pl.ds(base, G)], nrm_s.at[buf],
                            isem[buf]).wait()

    def issue_gathers(buf):
      for g in range(G):
        pltpu.async_copy(cur_h.at[src_v.at[buf, g]], rows_v.at[buf, g],
                         gsem[buf])

    def wait_gather(buf, g):
      pltpu.make_async_copy(cur_h.at[src_v.at[buf, g]], rows_v.at[buf, g],
                            gsem[buf]).wait()

    issue_idx(0, 0)
    wait_idx(0, 0)
    issue_gathers(0)
    issue_idx(1, 1)

    @pl.loop(0, nblk, step=2)
    def _(blk0):
      for half in range(2):
        blk = blk0 + half
        buf = half
        nbuf = 1 - half

        @pl.when(blk + 1 < nblk)
        def _():
          wait_idx(nbuf, blk + 1)
          issue_gathers(nbuf)

        handles = []
        for g in range(G):
          wait_gather(buf, g)

          @pl.loop(0, CHUNK, step=16)
          def _(e0, _g=g, _buf=buf):
            d16 = dst_v[_buf, _g, pl.ds(e0, 16)]
            n16 = nrm_s[_buf, _g, pl.ds(e0, 16)]
            dl = d16 - cid * hr
            ok = (dl >= 0) & (dl < hr)
            dst_v[_buf, _g, pl.ds(e0, 16)] = jnp.where(ok, dl, d16 & 16383)
            f16 = jnp.where(ok, n16, 0.0)
            for e in range(16):
              ns = _bcast16(f16, e)
              rows_v[_buf, _g, e0 + e, :] = rows_v[_buf, _g, e0 + e, :] * ns

          handles.append(
              pltpu.async_copy(rows_v.at[buf, g], acc.at[dst_v.at[buf, g]],
                               ssem[buf], add=True))
        for h in handles:
          h.wait()

        @pl.when(blk + 2 < nblk)
        def _():
          issue_idx(buf, blk + 2)

    plsc.subcore_barrier()
    for j in range(8):
      off = sid * stripe + j * zrows
      pltpu.sync_copy(acc.at[pl.ds(off, zrows)], zbuf)
      pltpu.sync_copy(zbuf, out_h.at[cid].at[pl.ds(off, zrows)])


  return round_kernel(cur, srcr, dstr, nrmr)


def _tc_mlp(x_p, W1, b1r, W2p, b2r, params, n_pad):
  bn = n_pad // 16
  f_in = x_p.shape[1]
  hid = W1.shape[1]

  def body(x_ref, w1_ref, b1_ref, w2_ref, b2_ref, p_ref, h_ref, acc_ref):
    xb = x_ref[...]
    z = jnp.dot(xb, w1_ref[...], preferred_element_type=jnp.float32)
    z = jnp.maximum(z + b1_ref[...], 0.0)
    h = jnp.dot(z, w2_ref[...], preferred_element_type=jnp.float32)
    h = h + b2_ref[...]
    wp = p_ref[0:1, :]
    bp = p_ref[1:2, 0:1]
    s = jnp.sum(h * wp, axis=1, keepdims=True) + bp
    r = jax.nn.sigmoid(s)
    h_ref[...] = h
    acc_ref[...] = r * h

  return pl.pallas_call(
      body,
      grid=(n_pad // bn,),
      in_specs=[
          pl.BlockSpec((bn, f_in), lambda i: (i, 0)),
          pl.BlockSpec((f_in, hid), lambda i: (0, 0)),
          pl.BlockSpec((1, hid), lambda i: (0, 0)),
          pl.BlockSpec((hid, L), lambda i: (0, 0)),
          pl.BlockSpec((1, L), lambda i: (0, 0)),
          pl.BlockSpec((2, L), lambda i: (0, 0)),
      ],
      out_specs=[
          pl.BlockSpec((bn, L), lambda i: (i, 0)),
          pl.BlockSpec((bn, L), lambda i: (i, 0)),
      ],
      out_shape=[
          jax.ShapeDtypeStruct((n_pad, L), jnp.float32),
          jax.ShapeDtypeStruct((n_pad, L), jnp.float32),
      ],
  )(x_p, W1, b1r, W2p, b2r, params)


def _tc_combine(parts, acc_in, params, n_pad, c_real, last):
  bn = n_pad // 16
  nxt = parts.reshape(n_pad, L)

  def body(p_ref, a_ref, prm_ref, aout_ref):
    h = p_ref[...]
    wp = prm_ref[0:1, :]
    bp = prm_ref[1:2, 0:1]
    s = jnp.sum(h * wp, axis=1, keepdims=True) + bp
    r = jax.nn.sigmoid(s)
    a = a_ref[...] + r * h
    if last:
      ch = lax.broadcasted_iota(jnp.int32, (bn, L), 1)
      mask = ch < c_real
      am = jnp.where(mask, a, -1e30)
      m = jnp.max(am, axis=1, keepdims=True)
      ex = jnp.where(mask, jnp.exp(a - m), 0.0)
      lse = jnp.log(jnp.sum(ex, axis=1, keepdims=True))
      aout_ref[...] = a - m - lse
    else:
      aout_ref[...] = a

  return pl.pallas_call(
      body,
      grid=(n_pad // bn,),
      in_specs=[
          pl.BlockSpec((bn, L), lambda i: (i, 0)),
          pl.BlockSpec((bn, L), lambda i: (i, 0)),
          pl.BlockSpec((2, L), lambda i: (0, 0)),
      ],
      out_specs=[
          pl.BlockSpec((bn, L), lambda i: (i, 0)),
      ],
      out_shape=[
          jax.ShapeDtypeStruct((n_pad, L), jnp.float32),
      ],
  )(nxt, acc_in, params)[0]


def kernel(x, edge_index, norm, W1, b1, W2, b2, Wp, bp):
  n, f_in = x.shape
  e = edge_index.shape[1]
  hid = W1.shape[1]
  c = W2.shape[1]

  n_pad = -(-(n + 64) // (NSUB * 8 * 8)) * (NSUB * 8 * 8)
  cpt = -(-e // (NSUB * CHUNK * 2 * G)) * (2 * G)
  e_pad = NSUB * cpt * CHUNK
  pad = e_pad - e

  src = jnp.concatenate([edge_index[0], jnp.zeros((pad,), jnp.int32)])
  dst = jnp.concatenate(
      [edge_index[1], n + (jnp.arange(pad, dtype=jnp.int32) % 64)])
  nrm = jnp.concatenate([norm, jnp.zeros((pad,), jnp.float32)])
  srcr = src.reshape(NSUB * cpt, CHUNK)
  dstr = dst.reshape(NSUB * cpt, CHUNK)
  nrmr = nrm.reshape(NSUB * cpt, CHUNK)

  x_p = jnp.concatenate([x, jnp.zeros((n_pad - n, f_in), jnp.float32)])
  W2p = jnp.concatenate([W2, jnp.zeros((hid, L - c), jnp.float32)], axis=1)
  b2r = jnp.concatenate([b2, jnp.zeros((L - c,), jnp.float32)]).reshape(1, L)
  b1r = b1.reshape(1, hid)
  wp_pad = jnp.concatenate([Wp[:, 0], jnp.zeros((L - c,), jnp.float32)])
  bp_row = jnp.concatenate([bp, jnp.zeros((L - 1,), jnp.float32)])
  params = jnp.stack([wp_pad, bp_row])

  cur, acc = _tc_mlp(x_p, W1, b1r, W2p, b2r, params, n_pad)
  for r in range(KSTEPS):
    parts = _sc_round(cur, srcr, dstr, nrmr, n_pad=n_pad, cpt=cpt)
    cur = parts.reshape(n_pad, L)
    acc = _tc_combine(parts, acc, params, n_pad, c,
                      last=(r == KSTEPS - 1))
  return acc[:n, :c]
